# baseline (speedup 1.0000x reference)
import jax
import jax.numpy as jnp
from jax.experimental import pallas as pl
from jax.experimental.pallas import tpu as pltpu

_K = 8
_ROWS_PER_BLOCK = 1024


def _select_block(x_ref, y_ref, out_ref):
    labels = x_ref[:, :]
    mask = (labels >= 0) & (labels < _K)
    out_ref[:, :] = jnp.where(mask, y_ref[:, :], 0.0)


def kernel(x, y):
    n, d = y.shape
    r = _ROWS_PER_BLOCK
    grid = n // r
    x2 = x.astype(jnp.int32).reshape(n, 1)
    return pl.pallas_call(
        _select_block,
        grid=(grid,),
        in_specs=[
            pl.BlockSpec((r, 1), lambda i: (i, 0)),
            pl.BlockSpec((r, d), lambda i: (i, 0)),
        ],
        out_specs=pl.BlockSpec((r, d), lambda i: (i, 0)),
        out_shape=jax.ShapeDtypeStruct((n, d), y.dtype),
        compiler_params=pltpu.CompilerParams(
            dimension_semantics=("arbitrary",),
        ),
    )(x2, y)
